# baseline (speedup 1.0000x reference)
import functools

import jax
import jax.numpy as jnp
from jax import lax
from jax.experimental import pallas as pl
from jax.experimental.pallas import tpu as pltpu
from jax.experimental.pallas import tpu_sc as plsc

CH = 128
K = 4


@functools.lru_cache(maxsize=None)
def _make_lookup(n_b: int, n_l: int, vocab: int, emb: int):
    info = plsc.get_sparse_core_info()
    nc, ns = info.num_cores, info.num_subcores
    nw = nc * ns
    n_tok = n_b * n_l
    nch = n_tok // CH // nw
    groups = nch // K
    cpr = n_b // CH
    ntd = emb // 8
    assert n_tok % (CH * nw * K) == 0 and n_b % CH == 0 and emb % 8 == 0

    mesh = plsc.VectorSubcoreMesh(core_axis_name="c", subcore_axis_name="s")

    @functools.partial(
        pl.kernel,
        mesh=mesh,
        out_type=jax.ShapeDtypeStruct((n_l, ntd, cpr, 8, CH), jnp.float32),
        compiler_params=pltpu.CompilerParams(
            use_tc_tiling_on_sc=False, needs_layout_passes=False),
        scratch_types=(
            [pltpu.VMEM((nch, CH), jnp.int32),
             pltpu.VMEM((K, CH, emb), jnp.float32),
             pltpu.VMEM((K, ntd, 8, CH), jnp.float32)]
            + [pltpu.SemaphoreType.DMA] * (2 * K)
        ),
    )
    def lookup(ids_hbm, table_hbm, out_hbm, idx_v, rows_v, rows_t, *sems):
        sg, so = sems[:K], sems[K:]
        wid = lax.axis_index("s") * nc + lax.axis_index("c")
        cbase = wid * nch
        pltpu.sync_copy(ids_hbm.at[pl.ds(cbase, nch)], idx_v)

        iota16 = lax.iota(jnp.int32, 16)
        d0s = tuple(range(0, emb, 16))
        rvec = [((d0 + iota16) >> 3).astype(jnp.int32) for d0 in d0s]
        svec = [((d0 + iota16) & 7).astype(jnp.int32) for d0 in d0s]

        def fire_gather(j, b):
            return pltpu.async_copy(
                table_hbm.at[idx_v.at[j]], rows_v.at[b], sg[b])

        def wait_gather(j, b):
            pltpu.make_async_copy(
                table_hbm.at[idx_v.at[j]], rows_v.at[b], sg[b]).wait()

        def wait_out(j, b):
            gid = cbase + j
            pltpu.make_async_copy(
                rows_t.at[b], out_hbm.at[gid // cpr, :, gid % cpr], so[b]
            ).wait()

        def transpose_slot(b):
            def body(i, carry):
                cvec = jnp.full((16,), i, jnp.int32)
                for t in range(len(d0s)):
                    val = rows_v[b, i, pl.ds(d0s[t], 16)]
                    plsc.store_scatter(
                        rows_t.at[b], [rvec[t], svec[t], cvec], val)
                return carry
            lax.fori_loop(0, CH, body, 0)

        def fire_out(j, b):
            gid = cbase + j
            l = gid // cpr
            c0 = gid % cpr
            return pltpu.async_copy(
                rows_t.at[b], out_hbm.at[l, :, c0], so[b])

        def process(g, first):
            for b in range(K):
                j = g * K + b
                wait_gather(j, b)
                if not first:
                    wait_out(j - K, b)
                transpose_slot(b)
                fire_out(j, b)
                fire_gather(jnp.minimum(j + K, nch - 1), b)

        for b in range(K):
            fire_gather(b, b)
        process(0, True)

        def loop_body(g, carry):
            process(g, False)
            return carry
        lax.fori_loop(1, groups, loop_body, 0)

        for b in range(K):
            wait_gather(nch - 1, b)
            wait_out((groups - 1) * K + b, b)

    return lookup


def kernel(token_ids, embed_weight):
    n_b, n_l = token_ids.shape
    vocab, emb = embed_weight.shape
    ids2d = jnp.transpose(token_ids).reshape((n_b * n_l) // CH, CH)
    out5 = _make_lookup(n_b, n_l, vocab, emb)(ids2d, embed_weight)
    return out5.transpose(2, 4, 0, 1, 3).reshape(n_b, n_l, emb)
